# baseline (speedup 1.0000x reference)
import jax
import jax.numpy as jnp
from jax import lax
from jax.experimental import pallas as pl
from jax.experimental.pallas import tpu as pltpu
from jax.experimental.pallas import tpu_sc as plsc

SEQ_OUT = 49
BATCH = 1024
VOCAB = 1000
EMB_ROWS = 100
EMB_DIM = 10
EMB_PAD = 128

NUM_SC = 2
NUM_SUBCORES = 16
NUM_WORKERS = NUM_SC * NUM_SUBCORES
B_PER_W = BATCH // NUM_WORKERS


def _sc_gather_body(table_hbm, idx_hbm, out_hbm, idx_v, rows_v, sem):
    wid = lax.axis_index("s") * NUM_SC + lax.axis_index("c")
    base = wid * B_PER_W
    pltpu.sync_copy(idx_hbm.at[pl.ds(base, B_PER_W)], idx_v)
    pltpu.async_copy(table_hbm.at[idx_v], rows_v, sem).wait()
    pltpu.sync_copy(rows_v, out_hbm.at[pl.ds(base, B_PER_W)])


def _sc_gather(emb_pad, idx):
    gather = pl.kernel(
        _sc_gather_body,
        out_type=jax.ShapeDtypeStruct((BATCH, EMB_PAD), jnp.float32),
        mesh=plsc.VectorSubcoreMesh(core_axis_name="c", subcore_axis_name="s"),
        scratch_types=[
            pltpu.VMEM((B_PER_W,), jnp.int32),
            pltpu.VMEM((B_PER_W, EMB_PAD), jnp.float32),
            pltpu.SemaphoreType.DMA,
        ],
    )
    return gather(emb_pad, idx)


def _proj_kernel(pooled_ref, w_ref, b_ref, out_ref):
    out = jax.lax.dot_general(
        pooled_ref[:, :], w_ref[:, :],
        dimension_numbers=(((1,), (0,)), ((), ())),
        preferred_element_type=jnp.float32,
    )
    out_ref[:, :] = out + b_ref[:, :]


def kernel(question, answer, emb_table, lin_w, lin_b):
    del question
    idx = answer[0].astype(jnp.int32)
    emb_pad = jnp.zeros((EMB_ROWS, EMB_PAD), jnp.float32).at[:, :EMB_DIM].set(
        emb_table
    )
    w_pad = jnp.zeros((EMB_PAD, VOCAB), jnp.float32).at[:EMB_DIM, :].set(
        lin_w.T
    )
    b2 = lin_b.reshape(1, VOCAB)

    pooled = _sc_gather(emb_pad, idx)

    tile = pl.pallas_call(
        _proj_kernel,
        out_shape=jax.ShapeDtypeStruct((BATCH, VOCAB), jnp.float32),
    )(pooled, w_pad, b2)

    return jnp.broadcast_to(tile[None], (SEQ_OUT, BATCH, VOCAB))
